# baseline (speedup 1.0000x reference)
import functools

import jax
import jax.numpy as jnp
from jax import lax
from jax.experimental import pallas as pl
from jax.experimental.pallas import tpu as pltpu
from jax.experimental.pallas import tpu_sc as plsc

DIM = 32
NENT = 1000000
NRELR = 1000
B = 16384
NC = 2
NS = 16
NW = NC * NS
L = 16

NBLK_FULL = NENT // 128
TAIL = NENT - NBLK_FULL * 128
PROWS = NENT // 8
PROWS_PAD = PROWS + 16
BLK_PER_W = 244
SLOTS = 246

BPW = B // NW
CHUNK2 = 64
NCH = BPW // CHUNK2
GPC = CHUNK2 // L


def _rsqrt16(x):
    xi = plsc.bitcast(x, jnp.int32)
    one = jnp.full((L,), 1, jnp.int32)
    yi = jnp.full((L,), 0x5F3759DF, jnp.int32) - lax.shift_right_arithmetic(xi, one)
    y = plsc.bitcast(yi, jnp.float32)
    for _ in range(3):
        y = y * (1.5 - 0.5 * x * y * y)
    return jnp.minimum(y, 1e12)


def _pack_body(entT_h, tail_h, p_h, v0, v1, o0, o1, vtail,
               sr0, sr1, sw0, sw1):
    w = lax.axis_index("s") * NC + lax.axis_index("c")
    start = w * BLK_PER_W + jnp.minimum(w, 5)
    iota = lax.iota(jnp.int32, L)
    iota2 = iota * 2

    def blk(i):
        return jnp.minimum(start + i, NBLK_FULL - 1)

    def fire_read(i, vb, sr):
        coff = pl.multiple_of(blk(i) * 128, 128)
        pltpu.async_copy(entT_h.at[:, pl.ds(coff, 128)], vb, sr)

    def transpose_pack(vb, ob):
        for r in range(16):
            for k in range(8):
                e = jnp.full((L,), 8 * r + k, jnp.int32)
                ev = plsc.load_gather(vb, [iota2, e])
                od = plsc.load_gather(vb, [iota2 + 1, e])
                packed = plsc.pack(ev, od, format=plsc.PackFormat.INTERLEAVED)
                ob[r, pl.ds(16 * k, L)] = plsc.bitcast(packed, jnp.int32)

    fire_read(0, v0, sr0)
    fire_read(1, v1, sr1)
    pltpu.async_copy(o0, p_h.at[pl.ds(PROWS, 16)], sw0)
    pltpu.async_copy(o1, p_h.at[pl.ds(PROWS, 16)], sw1)

    def step(t, carry):
        for b, (vb, ob, sr, sw) in enumerate(
            ((v0, o0, sr0, sw0), (v1, o1, sr1, sw1))):
            i = 2 * t + b
            pltpu.make_async_copy(entT_h.at[:, pl.ds(0, 128)], vb, sr).wait()
            pltpu.make_async_copy(ob, p_h.at[pl.ds(PROWS, 16)], sw).wait()
            transpose_pack(vb, ob)
            rowoff = pl.multiple_of(blk(i) * 16, 8)
            pltpu.async_copy(ob, p_h.at[pl.ds(rowoff, 16)], sw)
            fire_read(i + 2, vb, sr)
        return carry

    lax.fori_loop(0, SLOTS // 2, step, 0)

    pltpu.make_async_copy(entT_h.at[:, pl.ds(0, 128)], v0, sr0).wait()
    pltpu.make_async_copy(entT_h.at[:, pl.ds(0, 128)], v1, sr1).wait()
    pltpu.make_async_copy(o0, p_h.at[pl.ds(PROWS, 16)], sw0).wait()
    pltpu.make_async_copy(o1, p_h.at[pl.ds(PROWS, 16)], sw1).wait()

    @pl.when(w == NW - 1)
    def _tail():
        pltpu.sync_copy(tail_h, vtail)
        for r in range(TAIL // 8):
            for k in range(8):
                e = 8 * r + k
                ev = plsc.load_gather(vtail, [iota * 128 + e])
                od = plsc.load_gather(vtail, [iota * 128 + 64 + e])
                packed = plsc.pack(ev, od, format=plsc.PackFormat.INTERLEAVED)
                o0[r, pl.ds(16 * k, L)] = plsc.bitcast(packed, jnp.int32)
        pltpu.sync_copy(o0.at[pl.ds(0, TAIL // 8)],
                        p_h.at[pl.ds(NBLK_FULL * 16, TAIL // 8)])


_pack_table = functools.partial(
    pl.kernel,
    mesh=plsc.VectorSubcoreMesh(core_axis_name="c", subcore_axis_name="s"),
    out_type=jax.ShapeDtypeStruct((PROWS_PAD, 128), jnp.int32),
    scratch_types=[
        pltpu.VMEM((DIM, 128), jnp.float32),
        pltpu.VMEM((DIM, 128), jnp.float32),
        pltpu.VMEM((16, 128), jnp.int32),
        pltpu.VMEM((16, 128), jnp.int32),
        pltpu.VMEM((DIM * TAIL,), jnp.float32),
        pltpu.SemaphoreType.DMA,
        pltpu.SemaphoreType.DMA,
        pltpu.SemaphoreType.DMA,
        pltpu.SemaphoreType.DMA,
    ],
    compiler_params=pltpu.CompilerParams(needs_layout_passes=False),
)(_pack_body)


def _score_body(h8_h, t8_h, a8_h, b8_h, hr_h, tr_h, ar_h, br_h, rr_h,
                rel_h, p_h, gold_h, negd_h,
                h8x, t8x, a8x, b8x, hrx, trx, arx, brx, rrx,
                hb0, tb0, ab0, bb0, hb1, tb1, ab1, bb1,
                relbuf, gout, nout, sem0, sem1):
    w = lax.axis_index("s") * NC + lax.axis_index("c")

    for src, dst in ((h8_h, h8x), (t8_h, t8x), (a8_h, a8x), (b8_h, b8x),
                     (hr_h, hrx), (tr_h, trx), (ar_h, arx), (br_h, brx),
                     (rr_h, rrx)):
        pltpu.sync_copy(src.at[w], dst)
    pltpu.sync_copy(rel_h, relbuf)

    bufs = ((hb0, tb0, ab0, bb0), (hb1, tb1, ab1, bb1))
    sems = (sem0, sem1)
    idxs = (h8x, t8x, a8x, b8x)

    def fire(c):
        cc, off = c // 2, 64 * (c % 2)
        group = bufs[c % 2]
        return [
            pltpu.async_copy(p_h.at[ix.at[cc, pl.ds(off, CHUNK2)]], buf,
                             sems[c % 2])
            for ix, buf in zip(idxs, group)
        ]

    iota = lax.iota(jnp.int32, L)
    three = jnp.full((L,), 7, jnp.int32)
    pend = fire(0)

    for c in range(NCH):
        for cp in pend:
            cp.wait()
        if c + 1 < NCH:
            pend = fire(c + 1)
        hb, tb, ab, bb = bufs[c % 2]
        cc, off = c // 2, 64 * (c % 2)

        def group(g, carry):
            sl = pl.ds(off + g * L, L)
            row = jnp.full((L,), g * L, jnp.int32) + iota
            sh = (hrx[cc, sl] & three) * 16
            st = (trx[cc, sl] & three) * 16
            sa = (arx[cc, sl] & three) * 16
            sb = (brx[cc, sl] & three) * 16
            rid = rrx[cc, sl]
            z = jnp.zeros((L,), jnp.float32)
            hh = tt = rr = hr = ht = rt = z
            aa = bb_ = ar = ab_ = br = z
            for j in range(DIM // 2):
                jv = jnp.full((L,), j, jnp.int32)
                h0, h1 = plsc.unpack(
                    plsc.bitcast(plsc.load_gather(hb, [row, sh + jv]),
                                 jnp.bfloat16),
                    format=plsc.PackFormat.INTERLEAVED)
                t0, t1 = plsc.unpack(
                    plsc.bitcast(plsc.load_gather(tb, [row, st + jv]),
                                 jnp.bfloat16),
                    format=plsc.PackFormat.INTERLEAVED)
                a0, a1 = plsc.unpack(
                    plsc.bitcast(plsc.load_gather(ab, [row, sa + jv]),
                                 jnp.bfloat16),
                    format=plsc.PackFormat.INTERLEAVED)
                b0, b1 = plsc.unpack(
                    plsc.bitcast(plsc.load_gather(bb, [row, sb + jv]),
                                 jnp.bfloat16),
                    format=plsc.PackFormat.INTERLEAVED)
                r0 = plsc.load_gather(relbuf, [rid + (2 * j) * NRELR])
                r1 = plsc.load_gather(relbuf, [rid + (2 * j + 1) * NRELR])
                for h, t, a, b_, r in ((h0, t0, a0, b0, r0),
                                       (h1, t1, a1, b1, r1)):
                    hh += h * h
                    tt += t * t
                    rr += r * r
                    hr += h * r
                    ht += h * t
                    rt += r * t
                    aa += a * a
                    bb_ += b_ * b_
                    ar += a * r
                    ab_ += a * b_
                    br += b_ * r
            ih = _rsqrt16(hh)
            it = _rsqrt16(tt)
            g2 = hh * ih * ih + rr + tt * it * it + 2.0 * (
                hr * ih - ht * (ih * it) - rt * it)
            g2 = jnp.maximum(g2, 0.0)
            gval = g2 * _rsqrt16(g2)
            ia = _rsqrt16(aa)
            ib = _rsqrt16(bb_)
            n2 = aa * ia * ia + rr + bb_ * ib * ib + 2.0 * (
                ar * ia - ab_ * (ia * ib) - br * ib)
            n2 = jnp.maximum(n2, 0.0)
            nval = n2 * _rsqrt16(n2)
            gout[cc, sl] = -gval
            nout[cc, sl] = -nval
            return carry

        lax.fori_loop(0, GPC, group, 0)

    pltpu.sync_copy(gout, gold_h.at[w])
    pltpu.sync_copy(nout, negd_h.at[w])


_score = functools.partial(
    pl.kernel,
    mesh=plsc.VectorSubcoreMesh(core_axis_name="c", subcore_axis_name="s"),
    out_type=(
        jax.ShapeDtypeStruct((NW, 4, 128), jnp.float32),
        jax.ShapeDtypeStruct((NW, 4, 128), jnp.float32),
    ),
    scratch_types=(
        [pltpu.VMEM((4, 128), jnp.int32)] * 9
        + [pltpu.VMEM((CHUNK2, 128), jnp.int32)] * 8
        + [
            pltpu.VMEM((DIM * NRELR,), jnp.float32),
            pltpu.VMEM((4, 128), jnp.float32),
            pltpu.VMEM((4, 128), jnp.float32),
            pltpu.SemaphoreType.DMA,
            pltpu.SemaphoreType.DMA,
        ]
    ),
    compiler_params=pltpu.CompilerParams(needs_layout_passes=False),
)(_score_body)


def kernel(heads, tails, negative_heads, negative_tails, relations, ent_emb, rel_emb):
    def prep(ix):
        return ix.astype(jnp.int32).reshape(NW, 4, 128)

    entT = ent_emb.T
    tail = ent_emb[NBLK_FULL * 128:].T.reshape(-1)
    packed = _pack_table(entT, tail)

    relF = rel_emb.T.reshape(-1)
    ent_idx = [heads, tails, negative_heads, negative_tails]
    q8 = [prep(ix.astype(jnp.int32) >> 3) for ix in ent_idx]
    raw = [prep(ix) for ix in ent_idx] + [prep(relations)]
    gold, negd = _score(*q8, *raw, relF, packed)
    return gold.reshape(B), negd.reshape(B)
